# Initial kernel scaffold; baseline (speedup 1.0000x reference)
import jax
import jax.numpy as jnp
from jax.experimental import pallas as pl


def kernel(features, edges, pos_edges, neg_edges, Wg1, bg1, Wf1, bf1, Wf2, bf2, Wc1, bc1, Wmu, bmu, Wls, bls, Wl1, bl1, Wl2, bl2, emb_w):
    raise NotImplementedError("write your pallas kernel here")


# speedup vs baseline: 22.2027x; 22.2027x over previous
import functools

import jax
import jax.numpy as jnp
from jax import lax
from jax.experimental import pallas as pl
from jax.experimental.pallas import tpu as pltpu
from jax.experimental.pallas import tpu_sc as plsc

G, NG, EG, F = 32, 1024, 16384, 128
D2 = 32
DG = 64
NN = 32
P = 512

NC, NS = 2, 16
GPC = G // NC
EPT = EG // NS
NCH = EPT // 128
RPT = NG // NS

_MESH = dict(core_axis_name="c", subcore_axis_name="s", num_cores=NC,
             num_subcores=NS)


def _sc_counts_body(er, ones_hbm, z_hbm, cd_out, cs_out,
                    idx_d, idx_s, ones_v, zbuf, accd, accs):
    c = lax.axis_index("c")
    s = lax.axis_index("s")
    base = s * RPT
    pltpu.sync_copy(ones_hbm, ones_v)
    pltpu.sync_copy(z_hbm, zbuf)
    for gi in range(GPC):
        g = c * GPC + gi
        pltpu.sync_copy(zbuf, accd.at[pl.ds(base, RPT)])
        pltpu.sync_copy(zbuf, accs.at[pl.ds(base, RPT)])
        pltpu.sync_copy(er.at[g, 1, s], idx_d)
        pltpu.sync_copy(er.at[g, 0, s], idx_s)
        plsc.subcore_barrier()
        for j in range(NCH):
            pltpu.sync_copy(ones_v, accd.at[idx_d.at[j]], add=True)
            pltpu.sync_copy(ones_v, accs.at[idx_s.at[j]], add=True)
        plsc.subcore_barrier()
        pltpu.sync_copy(accd.at[pl.ds(base, RPT)],
                        cd_out.at[g, pl.ds(base, RPT)])
        pltpu.sync_copy(accs.at[pl.ds(base, RPT)],
                        cs_out.at[g, pl.ds(base, RPT)])
        plsc.subcore_barrier()


def _sc_counts(er):
    mesh = plsc.VectorSubcoreMesh(**_MESH)
    ones = jnp.ones((128, 128), jnp.float32)
    zeros = jnp.zeros((RPT, 128), jnp.float32)
    fn = pl.kernel(
        _sc_counts_body,
        out_type=(jax.ShapeDtypeStruct((G, NG, 128), jnp.float32),
                  jax.ShapeDtypeStruct((G, NG, 128), jnp.float32)),
        mesh=mesh,
        scratch_types=[
            pltpu.VMEM((NCH, 128), jnp.int32),
            pltpu.VMEM((NCH, 128), jnp.int32),
            pltpu.VMEM((128, 128), jnp.float32),
            pltpu.VMEM((RPT, 128), jnp.float32),
            pltpu.VMEM_SHARED((NG, 128), jnp.float32),
            pltpu.VMEM_SHARED((NG, 128), jnp.float32),
        ],
    )
    return fn(er, ones, zeros)


def _sc_scatter_body(width, table, gidx, sidx, z_hbm, out,
                     idx_g, idx_s, rb0, rb1, zbuf, acc, sem):
    c = lax.axis_index("c")
    s = lax.axis_index("s")
    base = s * RPT
    pltpu.sync_copy(z_hbm, zbuf)
    rbs = (rb0, rb1)
    for gi in range(GPC):
        g = c * GPC + gi
        pltpu.sync_copy(zbuf, acc.at[pl.ds(base, RPT)])
        pltpu.sync_copy(gidx.at[g, s], idx_g)
        pltpu.sync_copy(sidx.at[g, s], idx_s)
        plsc.subcore_barrier()
        cp = pltpu.async_copy(table.at[idx_g.at[0]], rbs[0], sem)
        for j in range(NCH):
            if j + 1 < NCH:
                cp_next = pltpu.async_copy(table.at[idx_g.at[j + 1]],
                                           rbs[(j + 1) % 2], sem)
            cp.wait()
            pltpu.sync_copy(rbs[j % 2], acc.at[idx_s.at[j]], add=True)
            if j + 1 < NCH:
                cp = cp_next
        plsc.subcore_barrier()
        pltpu.sync_copy(acc.at[pl.ds(base, RPT)],
                        out.at[g, pl.ds(base, RPT)])
        plsc.subcore_barrier()


def _sc_scatter(table, gidx, sidx, width):
    mesh = plsc.VectorSubcoreMesh(**_MESH)
    zeros = jnp.zeros((RPT, width), jnp.float32)
    fn = pl.kernel(
        functools.partial(_sc_scatter_body, width),
        out_type=jax.ShapeDtypeStruct((G, NG, width), jnp.float32),
        mesh=mesh,
        scratch_types=[
            pltpu.VMEM((NCH, 128), jnp.int32),
            pltpu.VMEM((NCH, 128), jnp.int32),
            pltpu.VMEM((128, width), jnp.float32),
            pltpu.VMEM((128, width), jnp.float32),
            pltpu.VMEM((RPT, width), jnp.float32),
            pltpu.VMEM_SHARED((NG, width), jnp.float32),
            pltpu.SemaphoreType.DMA,
        ],
    )
    return fn(table, gidx, sidx, zeros)


def _dinv2_col(cs_blk):
    col = cs_blk[:, :1]
    return jnp.where(col > 0.0, lax.rsqrt(jnp.maximum(col, 1.0)), 0.0)


def _tc_prep_body(feat, Wg1, cd, er, hs_o, srcg_o, dstg_o):
    g = pl.program_id(0)
    h = jnp.dot(feat[0], Wg1[...], preferred_element_type=jnp.float32)
    dinv = lax.rsqrt(cd[0][:, :1] + 1.0)
    hs_o[0] = h * dinv
    off = g * NG
    srcg_o[0] = er[0, 0] + off
    dstg_o[0] = er[0, 1] + off


def _tc_prep(features, Wg1, cd, er):
    return pl.pallas_call(
        _tc_prep_body,
        grid=(G,),
        in_specs=[
            pl.BlockSpec((1, NG, F), lambda g: (g, 0, 0)),
            pl.BlockSpec((F, F), lambda g: (0, 0)),
            pl.BlockSpec((1, NG, 128), lambda g: (g, 0, 0)),
            pl.BlockSpec((1, 2, NS, NCH, 128), lambda g: (g, 0, 0, 0, 0)),
        ],
        out_specs=[
            pl.BlockSpec((1, NG, F), lambda g: (g, 0, 0)),
            pl.BlockSpec((1, NS, NCH, 128), lambda g: (g, 0, 0, 0)),
            pl.BlockSpec((1, NS, NCH, 128), lambda g: (g, 0, 0, 0)),
        ],
        out_shape=[
            jax.ShapeDtypeStruct((G, NG, F), jnp.float32),
            jax.ShapeDtypeStruct((G, NS, NCH, 128), jnp.int32),
            jax.ShapeDtypeStruct((G, NS, NCH, 128), jnp.int32),
        ],
    )(features, Wg1, cd, er)


def _tc_mid_body(scat, hs, cd, cs, bg1, Wf1, bf1, Wf2, bf2,
                 S_o, Ssc_o, E0_o):
    dinv = lax.rsqrt(cd[0][:, :1] + 1.0)
    nf2 = (scat[0] + hs[0]) * dinv + bg1[...]
    a1 = jnp.tanh(jnp.dot(nf2, Wf1[...], preferred_element_type=jnp.float32)
                  + bf1[...])
    logits = (jnp.dot(a1, Wf2[...], preferred_element_type=jnp.float32)
              + bf2[...])
    S = jax.nn.softmax(logits, axis=1)
    S_o[0] = S
    Ssc_o[0] = jnp.concatenate(
        [S * _dinv2_col(cs[0]), jnp.zeros((NG, F - D2), jnp.float32)], axis=1)
    E0_o[0] = jnp.sum(nf2, axis=0, keepdims=True) * (1.0 / 32.0)


def _tc_mid(scat, hs, cd, cs, bg1, Wf1, bf1, Wf2, bf2):
    return pl.pallas_call(
        _tc_mid_body,
        grid=(G,),
        in_specs=[
            pl.BlockSpec((1, NG, F), lambda g: (g, 0, 0)),
            pl.BlockSpec((1, NG, F), lambda g: (g, 0, 0)),
            pl.BlockSpec((1, NG, 128), lambda g: (g, 0, 0)),
            pl.BlockSpec((1, NG, 128), lambda g: (g, 0, 0)),
            pl.BlockSpec((1, F), lambda g: (0, 0)),
            pl.BlockSpec((F, F), lambda g: (0, 0)),
            pl.BlockSpec((1, F), lambda g: (0, 0)),
            pl.BlockSpec((F, D2), lambda g: (0, 0)),
            pl.BlockSpec((1, D2), lambda g: (0, 0)),
        ],
        out_specs=[
            pl.BlockSpec((1, NG, D2), lambda g: (g, 0, 0)),
            pl.BlockSpec((1, NG, F), lambda g: (g, 0, 0)),
            pl.BlockSpec((1, 1, F), lambda g: (g, 0, 0)),
        ],
        out_shape=[
            jax.ShapeDtypeStruct((G, NG, D2), jnp.float32),
            jax.ShapeDtypeStruct((G, NG, F), jnp.float32),
            jax.ShapeDtypeStruct((G, 1, F), jnp.float32),
        ],
    )(scat, hs, cd, cs, bg1, Wf1, bf1, Wf2, bf2)


def _tc_pen_body(scat2, S_in, cs, pen_o):
    Sg = S_in[0]
    lms = Sg - _dinv2_col(cs[0]) * scat2[0][:, :D2]
    na = lax.dot_general(Sg, lms, (((0,), (0,)), ((), ())),
                         preferred_element_type=jnp.float32)
    rs = jnp.sum(jnp.abs(na), axis=1, keepdims=True)
    eye = (lax.broadcasted_iota(jnp.int32, (D2, D2), 0)
           == lax.broadcasted_iota(jnp.int32, (D2, D2), 1))
    dcol = (jnp.sum(jnp.where(eye, na, 0.0), axis=1, keepdims=True)
            / jnp.maximum(rs, 1e-12))
    pen = jnp.sum(31.0 * dcol * dcol + (dcol - 1.0) ** 2) * (1.0 / 1024.0)
    pen_o[0] = jnp.full((1, 128), pen, jnp.float32)


def _tc_pen(scat2, S, cs):
    return pl.pallas_call(
        _tc_pen_body,
        grid=(G,),
        in_specs=[
            pl.BlockSpec((1, NG, F), lambda g: (g, 0, 0)),
            pl.BlockSpec((1, NG, D2), lambda g: (g, 0, 0)),
            pl.BlockSpec((1, NG, 128), lambda g: (g, 0, 0)),
        ],
        out_specs=[pl.BlockSpec((1, 1, 128), lambda g: (g, 0, 0))],
        out_shape=[jax.ShapeDtypeStruct((G, 1, 128), jnp.float32)],
    )(scat2, S, cs)[0]


def _tc_final_body(E0, pos_e, neg_e, Wc1, bc1, Wmu, bmu, Wls, bls,
                   Wl1, bl1, Wl2, bl2, emb_w, pen_vec,
                   loss_o, pen_o, pp_o, np_o):
    f32 = jnp.float32

    def onehot(col):
        return (col == lax.broadcasted_iota(jnp.int32, (P, NN), 1)).astype(f32)

    Os = onehot(pos_e[:, 0:1])
    Od = onehot(pos_e[:, 1:2])
    Ns = onehot(neg_e[:, 0:1])
    Nd = onehot(neg_e[:, 1:2])

    ones_col = jnp.ones((P, 1), f32)
    cnt = lax.dot_general(Od, ones_col, (((0,), (0,)), ((), ())),
                          preferred_element_type=f32)
    dinv = lax.rsqrt(cnt + 1.0)
    norm = (jnp.dot(Os, dinv, preferred_element_type=f32)
            * jnp.dot(Od, dinv, preferred_element_type=f32))
    d2 = dinv * dinv

    def gcn2(X, W, b):
        H = jnp.dot(X, W[...], preferred_element_type=f32)
        gath = jnp.dot(Os, H, preferred_element_type=f32)
        agg = lax.dot_general(Od, norm * gath, (((0,), (0,)), ((), ())),
                              preferred_element_type=f32)
        return agg + d2 * H + b[...]

    h1 = jax.nn.relu(gcn2(E0[...], Wc1, bc1))
    mu = gcn2(h1, Wmu, bmu)
    ls = jnp.minimum(gcn2(h1, Wls, bls), 10.0)

    emb_full = jnp.concatenate([emb_w[...], mu], axis=1)

    def pred(Oa, Ob):
        fx = jnp.dot(jnp.dot(Oa, emb_full, preferred_element_type=f32),
                     Wl1[...], preferred_element_type=f32) + bl1[...]
        fy = jnp.dot(jnp.dot(Ob, emb_full, preferred_element_type=f32),
                     Wl2[...], preferred_element_type=f32) + bl2[...]
        return jax.nn.sigmoid(-jnp.sum(fx * fy, axis=1, keepdims=True))

    ppred = pred(Os, Od)
    npred = pred(Ns, Nd)
    EPS = 1e-15
    rec = (-jnp.mean(jnp.log(ppred + EPS))
           - jnp.mean(jnp.log(1.0 - npred + EPS)))
    kl = -0.5 * jnp.sum(1.0 + 2.0 * ls - mu * mu - jnp.exp(2.0 * ls)) \
        * (1.0 / (NN * NN))
    loss_o[...] = jnp.full((1, 1), rec + kl, f32)
    pen_o[...] = jnp.full((1, 1), jnp.sum(pen_vec[:, :1]) * (1.0 / G), f32)
    pp_o[...] = ppred
    np_o[...] = npred


def _tc_final(E0, pos_e, neg_e, Wc1, bc1, Wmu, bmu, Wls, bls,
              Wl1, bl1, Wl2, bl2, emb_w, pen_vec):
    return pl.pallas_call(
        _tc_final_body,
        out_shape=[
            jax.ShapeDtypeStruct((1, 1), jnp.float32),
            jax.ShapeDtypeStruct((1, 1), jnp.float32),
            jax.ShapeDtypeStruct((P, 1), jnp.float32),
            jax.ShapeDtypeStruct((P, 1), jnp.float32),
        ],
    )(E0, pos_e, neg_e, Wc1, bc1, Wmu, bmu, Wls, bls,
      Wl1, bl1, Wl2, bl2, emb_w, pen_vec)


def kernel(features, edges, pos_edges, neg_edges, Wg1, bg1, Wf1, bf1,
           Wf2, bf2, Wc1, bc1, Wmu, bmu, Wls, bls, Wl1, bl1, Wl2, bl2,
           emb_w):
    er = edges.astype(jnp.int32).reshape(G, 2, NS, NCH, 128)

    cd, cs = _sc_counts(er)

    hs, srcg, dstg = _tc_prep(features, Wg1, cd, er)

    scat = _sc_scatter(hs.reshape(G * NG, F), srcg,
                       er[:, 1], width=F)

    S, Ssc, E0 = _tc_mid(scat, hs, cd, cs, bg1.reshape(1, F),
                         Wf1, bf1.reshape(1, F), Wf2, bf2.reshape(1, D2))
    E0 = E0.reshape(G, F)

    scat2 = _sc_scatter(Ssc.reshape(G * NG, F), dstg,
                        er[:, 0], width=F)

    pen_vec = _tc_pen(scat2, S, cs).reshape(G, 128)

    loss, pen, pp, npred = _tc_final(
        E0, pos_edges.astype(jnp.int32), neg_edges.astype(jnp.int32),
        Wc1, bc1.reshape(1, 2 * DG), Wmu, bmu.reshape(1, DG),
        Wls, bls.reshape(1, DG), Wl1, bl1.reshape(1, F),
        Wl2, bl2.reshape(1, F), emb_w, pen_vec)

    return (loss[0, 0], pen[0, 0], pp[:, 0], npred[:, 0])


# speedup vs baseline: 27.9309x; 1.2580x over previous
import functools

import jax
import jax.numpy as jnp
from jax import lax
from jax.experimental import pallas as pl
from jax.experimental.pallas import tpu as pltpu
from jax.experimental.pallas import tpu_sc as plsc

G, NG, EG, F = 32, 1024, 16384, 128
D2 = 32
DG = 64
NN = 32
P = 512

NC, NS = 2, 16
GPC = G // NC
EPT = EG // NS
NCH = EPT // 128
RPT = NG // NS

_MESH = dict(core_axis_name="c", subcore_axis_name="s", num_cores=NC,
             num_subcores=NS)


def _sc_counts_body(er, z_hbm, out, idx_v, cntd, cnts):
    c = lax.axis_index("c")
    s = lax.axis_index("s")
    one16 = jnp.ones((16,), jnp.float32)

    def body(gi):
        g = c * GPC + gi
        pltpu.sync_copy(z_hbm, cntd)
        pltpu.sync_copy(z_hbm, cnts)
        pltpu.sync_copy(er.at[g, 1, s], idx_v)
        for j in range(NCH):
            for k in range(8):
                idx = idx_v[j, pl.ds(k * 16, 16)]
                plsc.addupdate_scatter(cntd, [idx], one16)
        pltpu.sync_copy(er.at[g, 0, s], idx_v)
        for j in range(NCH):
            for k in range(8):
                idx = idx_v[j, pl.ds(k * 16, 16)]
                plsc.addupdate_scatter(cnts, [idx], one16)
        pltpu.sync_copy(cntd, out.at[g, 0, pl.ds(s * NG, NG)])
        pltpu.sync_copy(cnts, out.at[g, 1, pl.ds(s * NG, NG)])

    pl.loop(0, GPC)(body)


def _sc_counts(er):
    mesh = plsc.VectorSubcoreMesh(**_MESH)
    zeros = jnp.zeros((NG,), jnp.float32)
    fn = pl.kernel(
        _sc_counts_body,
        out_type=jax.ShapeDtypeStruct((G, 2, NS * NG), jnp.float32),
        mesh=mesh,
        scratch_types=[
            pltpu.VMEM((NCH, 128), jnp.int32),
            pltpu.VMEM((NG,), jnp.float32),
            pltpu.VMEM((NG,), jnp.float32),
        ],
        compiler_params=pltpu.CompilerParams(needs_layout_passes=False),
    )
    return fn(er, zeros)


def _sc_scatter_body(width, table, gidx, sidx, z_hbm, out,
                     idx_g, idx_s, rb0, rb1, zbuf, acc, sem):
    c = lax.axis_index("c")
    s = lax.axis_index("s")
    base = s * RPT
    pltpu.sync_copy(z_hbm, zbuf)
    rbs = (rb0, rb1)
    for gi in range(GPC):
        g = c * GPC + gi
        pltpu.sync_copy(zbuf, acc.at[pl.ds(base, RPT)])
        pltpu.sync_copy(gidx.at[g, s], idx_g)
        pltpu.sync_copy(sidx.at[g, s], idx_s)
        plsc.subcore_barrier()
        cp = pltpu.async_copy(table.at[idx_g.at[0]], rbs[0], sem)
        for j in range(NCH):
            if j + 1 < NCH:
                cp_next = pltpu.async_copy(table.at[idx_g.at[j + 1]],
                                           rbs[(j + 1) % 2], sem)
            cp.wait()
            pltpu.sync_copy(rbs[j % 2], acc.at[idx_s.at[j]], add=True)
            if j + 1 < NCH:
                cp = cp_next
        plsc.subcore_barrier()
        pltpu.sync_copy(acc.at[pl.ds(base, RPT)],
                        out.at[g, pl.ds(base, RPT)])
        plsc.subcore_barrier()


def _sc_scatter(table, gidx, sidx, width):
    mesh = plsc.VectorSubcoreMesh(**_MESH)
    zeros = jnp.zeros((RPT, width), jnp.float32)
    fn = pl.kernel(
        functools.partial(_sc_scatter_body, width),
        out_type=jax.ShapeDtypeStruct((G, NG, width), jnp.float32),
        mesh=mesh,
        scratch_types=[
            pltpu.VMEM((NCH, 128), jnp.int32),
            pltpu.VMEM((NCH, 128), jnp.int32),
            pltpu.VMEM((128, width), jnp.float32),
            pltpu.VMEM((128, width), jnp.float32),
            pltpu.VMEM((RPT, width), jnp.float32),
            pltpu.VMEM_SHARED((NG, width), jnp.float32),
            pltpu.SemaphoreType.DMA,
        ],
    )
    return fn(table, gidx, sidx, zeros)


def _cnt_col(part):
    ones = jnp.ones((NS, 1), jnp.float32)
    return lax.dot_general(part, ones, (((0,), (0,)), ((), ())),
                           preferred_element_type=jnp.float32)


def _dinv2_col(cs_part):
    col = _cnt_col(cs_part)
    return jnp.where(col > 0.0, lax.rsqrt(jnp.maximum(col, 1.0)), 0.0)


def _tc_prep_body(feat, Wg1, cp, er, hs_o, srcg_o, dstg_o):
    g = pl.program_id(0)
    h = jnp.dot(feat[0], Wg1[...], preferred_element_type=jnp.float32)
    dinv = lax.rsqrt(_cnt_col(cp[0, 0]) + 1.0)
    hs_o[0] = h * dinv
    off = g * NG
    srcg_o[0] = er[0, 0] + off
    dstg_o[0] = er[0, 1] + off


def _tc_prep(features, Wg1, cd, er):
    return pl.pallas_call(
        _tc_prep_body,
        grid=(G,),
        in_specs=[
            pl.BlockSpec((1, NG, F), lambda g: (g, 0, 0)),
            pl.BlockSpec((F, F), lambda g: (0, 0)),
            pl.BlockSpec((1, 2, NS, NG), lambda g: (g, 0, 0, 0)),
            pl.BlockSpec((1, 2, NS, NCH, 128), lambda g: (g, 0, 0, 0, 0)),
        ],
        out_specs=[
            pl.BlockSpec((1, NG, F), lambda g: (g, 0, 0)),
            pl.BlockSpec((1, NS, NCH, 128), lambda g: (g, 0, 0, 0)),
            pl.BlockSpec((1, NS, NCH, 128), lambda g: (g, 0, 0, 0)),
        ],
        out_shape=[
            jax.ShapeDtypeStruct((G, NG, F), jnp.float32),
            jax.ShapeDtypeStruct((G, NS, NCH, 128), jnp.int32),
            jax.ShapeDtypeStruct((G, NS, NCH, 128), jnp.int32),
        ],
    )(features, Wg1, cd, er)


def _tc_mid_body(scat, hs, cp, bg1, Wf1, bf1, Wf2, bf2,
                 S_o, Ssc_o, E0_o):
    dinv = lax.rsqrt(_cnt_col(cp[0, 0]) + 1.0)
    nf2 = (scat[0] + hs[0]) * dinv + bg1[...]
    a1 = jnp.tanh(jnp.dot(nf2, Wf1[...], preferred_element_type=jnp.float32)
                  + bf1[...])
    logits = (jnp.dot(a1, Wf2[...], preferred_element_type=jnp.float32)
              + bf2[...])
    S = jax.nn.softmax(logits, axis=1)
    S_o[0] = S
    Ssc_o[0] = jnp.concatenate(
        [S * _dinv2_col(cp[0, 1]), jnp.zeros((NG, F - D2), jnp.float32)], axis=1)
    E0_o[0] = jnp.sum(nf2, axis=0, keepdims=True) * (1.0 / 32.0)


def _tc_mid(scat, hs, cp, bg1, Wf1, bf1, Wf2, bf2):
    return pl.pallas_call(
        _tc_mid_body,
        grid=(G,),
        in_specs=[
            pl.BlockSpec((1, NG, F), lambda g: (g, 0, 0)),
            pl.BlockSpec((1, NG, F), lambda g: (g, 0, 0)),
            pl.BlockSpec((1, 2, NS, NG), lambda g: (g, 0, 0, 0)),
            pl.BlockSpec((1, F), lambda g: (0, 0)),
            pl.BlockSpec((F, F), lambda g: (0, 0)),
            pl.BlockSpec((1, F), lambda g: (0, 0)),
            pl.BlockSpec((F, D2), lambda g: (0, 0)),
            pl.BlockSpec((1, D2), lambda g: (0, 0)),
        ],
        out_specs=[
            pl.BlockSpec((1, NG, D2), lambda g: (g, 0, 0)),
            pl.BlockSpec((1, NG, F), lambda g: (g, 0, 0)),
            pl.BlockSpec((1, 1, F), lambda g: (g, 0, 0)),
        ],
        out_shape=[
            jax.ShapeDtypeStruct((G, NG, D2), jnp.float32),
            jax.ShapeDtypeStruct((G, NG, F), jnp.float32),
            jax.ShapeDtypeStruct((G, 1, F), jnp.float32),
        ],
    )(scat, hs, cp, bg1, Wf1, bf1, Wf2, bf2)


def _tc_pen_body(scat2, S_in, cp, pen_o):
    Sg = S_in[0]
    lms = Sg - _dinv2_col(cp[0, 1]) * scat2[0][:, :D2]
    na = lax.dot_general(Sg, lms, (((0,), (0,)), ((), ())),
                         preferred_element_type=jnp.float32)
    rs = jnp.sum(jnp.abs(na), axis=1, keepdims=True)
    eye = (lax.broadcasted_iota(jnp.int32, (D2, D2), 0)
           == lax.broadcasted_iota(jnp.int32, (D2, D2), 1))
    dcol = (jnp.sum(jnp.where(eye, na, 0.0), axis=1, keepdims=True)
            / jnp.maximum(rs, 1e-12))
    pen = jnp.sum(31.0 * dcol * dcol + (dcol - 1.0) ** 2) * (1.0 / 1024.0)
    pen_o[0] = jnp.full((1, 128), pen, jnp.float32)


def _tc_pen(scat2, S, cp):
    return pl.pallas_call(
        _tc_pen_body,
        grid=(G,),
        in_specs=[
            pl.BlockSpec((1, NG, F), lambda g: (g, 0, 0)),
            pl.BlockSpec((1, NG, D2), lambda g: (g, 0, 0)),
            pl.BlockSpec((1, 2, NS, NG), lambda g: (g, 0, 0, 0)),
        ],
        out_specs=[pl.BlockSpec((1, 1, 128), lambda g: (g, 0, 0))],
        out_shape=[jax.ShapeDtypeStruct((G, 1, 128), jnp.float32)],
    )(scat2, S, cp)[0]


def _tc_final_body(E0, pos_e, neg_e, Wc1, bc1, Wmu, bmu, Wls, bls,
                   Wl1, bl1, Wl2, bl2, emb_w, pen_vec,
                   loss_o, pen_o, pp_o, np_o):
    f32 = jnp.float32

    def onehot(col):
        return (col == lax.broadcasted_iota(jnp.int32, (P, NN), 1)).astype(f32)

    Os = onehot(pos_e[:, 0:1])
    Od = onehot(pos_e[:, 1:2])
    Ns = onehot(neg_e[:, 0:1])
    Nd = onehot(neg_e[:, 1:2])

    ones_col = jnp.ones((P, 1), f32)
    cnt = lax.dot_general(Od, ones_col, (((0,), (0,)), ((), ())),
                          preferred_element_type=f32)
    dinv = lax.rsqrt(cnt + 1.0)
    norm = (jnp.dot(Os, dinv, preferred_element_type=f32)
            * jnp.dot(Od, dinv, preferred_element_type=f32))
    d2 = dinv * dinv

    def gcn2(X, W, b):
        H = jnp.dot(X, W[...], preferred_element_type=f32)
        gath = jnp.dot(Os, H, preferred_element_type=f32)
        agg = lax.dot_general(Od, norm * gath, (((0,), (0,)), ((), ())),
                              preferred_element_type=f32)
        return agg + d2 * H + b[...]

    h1 = jax.nn.relu(gcn2(E0[...], Wc1, bc1))
    mu = gcn2(h1, Wmu, bmu)
    ls = jnp.minimum(gcn2(h1, Wls, bls), 10.0)

    emb_full = jnp.concatenate([emb_w[...], mu], axis=1)

    def pred(Oa, Ob):
        fx = jnp.dot(jnp.dot(Oa, emb_full, preferred_element_type=f32),
                     Wl1[...], preferred_element_type=f32) + bl1[...]
        fy = jnp.dot(jnp.dot(Ob, emb_full, preferred_element_type=f32),
                     Wl2[...], preferred_element_type=f32) + bl2[...]
        return jax.nn.sigmoid(-jnp.sum(fx * fy, axis=1, keepdims=True))

    ppred = pred(Os, Od)
    npred = pred(Ns, Nd)
    EPS = 1e-15
    rec = (-jnp.mean(jnp.log(ppred + EPS))
           - jnp.mean(jnp.log(1.0 - npred + EPS)))
    kl = -0.5 * jnp.sum(1.0 + 2.0 * ls - mu * mu - jnp.exp(2.0 * ls)) \
        * (1.0 / (NN * NN))
    loss_o[...] = jnp.full((1, 1), rec + kl, f32)
    pen_o[...] = jnp.full((1, 1), jnp.sum(pen_vec[:, :1]) * (1.0 / G), f32)
    pp_o[...] = ppred
    np_o[...] = npred


def _tc_final(E0, pos_e, neg_e, Wc1, bc1, Wmu, bmu, Wls, bls,
              Wl1, bl1, Wl2, bl2, emb_w, pen_vec):
    return pl.pallas_call(
        _tc_final_body,
        out_shape=[
            jax.ShapeDtypeStruct((1, 1), jnp.float32),
            jax.ShapeDtypeStruct((1, 1), jnp.float32),
            jax.ShapeDtypeStruct((P, 1), jnp.float32),
            jax.ShapeDtypeStruct((P, 1), jnp.float32),
        ],
    )(E0, pos_e, neg_e, Wc1, bc1, Wmu, bmu, Wls, bls,
      Wl1, bl1, Wl2, bl2, emb_w, pen_vec)


def kernel(features, edges, pos_edges, neg_edges, Wg1, bg1, Wf1, bf1,
           Wf2, bf2, Wc1, bc1, Wmu, bmu, Wls, bls, Wl1, bl1, Wl2, bl2,
           emb_w):
    er = edges.astype(jnp.int32).reshape(G, 2, NS, NCH, 128)

    cp = _sc_counts(er).reshape(G, 2, NS, NG)

    hs, srcg, dstg = _tc_prep(features, Wg1, cp, er)

    scat = _sc_scatter(hs.reshape(G * NG, F), srcg,
                       er[:, 1], width=F)

    S, Ssc, E0 = _tc_mid(scat, hs, cp, bg1.reshape(1, F),
                         Wf1, bf1.reshape(1, F), Wf2, bf2.reshape(1, D2))
    E0 = E0.reshape(G, F)

    scat2 = _sc_scatter(Ssc.reshape(G * NG, F), dstg,
                        er[:, 0], width=F)

    pen_vec = _tc_pen(scat2, S, cp).reshape(G, 128)

    loss, pen, pp, npred = _tc_final(
        E0, pos_edges.astype(jnp.int32), neg_edges.astype(jnp.int32),
        Wc1, bc1.reshape(1, 2 * DG), Wmu, bmu.reshape(1, DG),
        Wls, bls.reshape(1, DG), Wl1, bl1.reshape(1, F),
        Wl2, bl2.reshape(1, F), emb_w, pen_vec)

    return (loss[0, 0], pen[0, 0], pp[:, 0], npred[:, 0])


# speedup vs baseline: 29.1352x; 1.0431x over previous
import functools

import jax
import jax.numpy as jnp
from jax import lax
from jax.experimental import pallas as pl
from jax.experimental.pallas import tpu as pltpu
from jax.experimental.pallas import tpu_sc as plsc

G, NG, EG, F = 32, 1024, 16384, 128
D2 = 32
DG = 64
NN = 32
P = 512

NC, NS = 2, 16
GPC = G // NC
EPT = EG // NS
NCH = EPT // 128
RPT = NG // NS

_MESH = dict(core_axis_name="c", subcore_axis_name="s", num_cores=NC,
             num_subcores=NS)


def _sc_counts_body(er, z_hbm, out, idx_v, cntd, cnts):
    c = lax.axis_index("c")
    s = lax.axis_index("s")
    one16 = jnp.ones((16,), jnp.float32)

    def body(gi):
        g = c * GPC + gi
        pltpu.sync_copy(z_hbm, cntd)
        pltpu.sync_copy(z_hbm, cnts)
        pltpu.sync_copy(er.at[g, 1, s], idx_v)
        for j in range(NCH):
            for k in range(8):
                idx = idx_v[j, pl.ds(k * 16, 16)]
                plsc.addupdate_scatter(cntd, [idx], one16)
        pltpu.sync_copy(er.at[g, 0, s], idx_v)
        for j in range(NCH):
            for k in range(8):
                idx = idx_v[j, pl.ds(k * 16, 16)]
                plsc.addupdate_scatter(cnts, [idx], one16)
        pltpu.sync_copy(cntd, out.at[g, 0, pl.ds(s * NG, NG)])
        pltpu.sync_copy(cnts, out.at[g, 1, pl.ds(s * NG, NG)])

    pl.loop(0, GPC)(body)


def _sc_counts(er):
    mesh = plsc.VectorSubcoreMesh(**_MESH)
    zeros = jnp.zeros((NG,), jnp.float32)
    fn = pl.kernel(
        _sc_counts_body,
        out_type=jax.ShapeDtypeStruct((G, 2, NS * NG), jnp.float32),
        mesh=mesh,
        scratch_types=[
            pltpu.VMEM((NCH, 128), jnp.int32),
            pltpu.VMEM((NG,), jnp.float32),
            pltpu.VMEM((NG,), jnp.float32),
        ],
        compiler_params=pltpu.CompilerParams(needs_layout_passes=False),
    )
    return fn(er, zeros)


def _sc_scatter_body(width, table, gidx, sidx, z_hbm, out,
                     idx_g, idx_s, rb0, rb1, rb2, zbuf, acc, semg, sems):
    c = lax.axis_index("c")
    s = lax.axis_index("s")
    base = s * RPT
    pltpu.sync_copy(z_hbm, zbuf)
    rbs = (rb0, rb1, rb2)
    for gi in range(GPC):
        g = c * GPC + gi
        pltpu.sync_copy(zbuf, acc.at[pl.ds(base, RPT)])
        pltpu.sync_copy(gidx.at[g, s], idx_g)
        pltpu.sync_copy(sidx.at[g, s], idx_s)
        plsc.subcore_barrier()
        cps = [None] * NCH
        scs = [None] * NCH
        cps[0] = pltpu.async_copy(table.at[idx_g.at[0]], rbs[0], semg)
        if NCH > 1:
            cps[1] = pltpu.async_copy(table.at[idx_g.at[1]], rbs[1], semg)
        for j in range(NCH):
            cps[j].wait()
            if j >= 1:
                scs[j - 1].wait()
            if j + 2 < NCH:
                cps[j + 2] = pltpu.async_copy(table.at[idx_g.at[j + 2]],
                                              rbs[(j + 2) % 3], semg)
            scs[j] = pltpu.async_copy(rbs[j % 3], acc.at[idx_s.at[j]],
                                      sems, add=True)
        scs[NCH - 1].wait()
        plsc.subcore_barrier()
        pltpu.sync_copy(acc.at[pl.ds(base, RPT)],
                        out.at[g, pl.ds(base, RPT)])
        plsc.subcore_barrier()


def _sc_scatter(table, gidx, sidx, width):
    mesh = plsc.VectorSubcoreMesh(**_MESH)
    zeros = jnp.zeros((RPT, width), jnp.float32)
    fn = pl.kernel(
        functools.partial(_sc_scatter_body, width),
        out_type=jax.ShapeDtypeStruct((G, NG, width), jnp.float32),
        mesh=mesh,
        scratch_types=[
            pltpu.VMEM((NCH, 128), jnp.int32),
            pltpu.VMEM((NCH, 128), jnp.int32),
            pltpu.VMEM((128, width), jnp.float32),
            pltpu.VMEM((128, width), jnp.float32),
            pltpu.VMEM((128, width), jnp.float32),
            pltpu.VMEM((RPT, width), jnp.float32),
            pltpu.VMEM_SHARED((NG, width), jnp.float32),
            pltpu.SemaphoreType.DMA,
            pltpu.SemaphoreType.DMA,
        ],
    )
    return fn(table, gidx, sidx, zeros)


def _cnt_col(part):
    ones = jnp.ones((NS, 1), jnp.float32)
    return lax.dot_general(part, ones, (((0,), (0,)), ((), ())),
                           preferred_element_type=jnp.float32)


def _dinv2_col(cs_part):
    col = _cnt_col(cs_part)
    return jnp.where(col > 0.0, lax.rsqrt(jnp.maximum(col, 1.0)), 0.0)


def _tc_prep_body(feat, Wg1, cp, er, hs_o, srcg_o, dstg_o):
    g = pl.program_id(0)
    h = jnp.dot(feat[0], Wg1[...], preferred_element_type=jnp.float32)
    dinv = lax.rsqrt(_cnt_col(cp[0, 0]) + 1.0)
    hs_o[0] = h * dinv
    off = g * NG
    srcg_o[0] = er[0, 0] + off
    dstg_o[0] = er[0, 1] + off


def _tc_prep(features, Wg1, cd, er):
    return pl.pallas_call(
        _tc_prep_body,
        grid=(G,),
        in_specs=[
            pl.BlockSpec((1, NG, F), lambda g: (g, 0, 0)),
            pl.BlockSpec((F, F), lambda g: (0, 0)),
            pl.BlockSpec((1, 2, NS, NG), lambda g: (g, 0, 0, 0)),
            pl.BlockSpec((1, 2, NS, NCH, 128), lambda g: (g, 0, 0, 0, 0)),
        ],
        out_specs=[
            pl.BlockSpec((1, NG, F), lambda g: (g, 0, 0)),
            pl.BlockSpec((1, NS, NCH, 128), lambda g: (g, 0, 0, 0)),
            pl.BlockSpec((1, NS, NCH, 128), lambda g: (g, 0, 0, 0)),
        ],
        out_shape=[
            jax.ShapeDtypeStruct((G, NG, F), jnp.float32),
            jax.ShapeDtypeStruct((G, NS, NCH, 128), jnp.int32),
            jax.ShapeDtypeStruct((G, NS, NCH, 128), jnp.int32),
        ],
    )(features, Wg1, cd, er)


def _tc_mid_body(scat, hs, cp, bg1, Wf1, bf1, Wf2, bf2,
                 S_o, Ssc_o, E0_o):
    dinv = lax.rsqrt(_cnt_col(cp[0, 0]) + 1.0)
    nf2 = (scat[0] + hs[0]) * dinv + bg1[...]
    a1 = jnp.tanh(jnp.dot(nf2, Wf1[...], preferred_element_type=jnp.float32)
                  + bf1[...])
    logits = (jnp.dot(a1, Wf2[...], preferred_element_type=jnp.float32)
              + bf2[...])
    S = jax.nn.softmax(logits, axis=1)
    S_o[0] = S
    Ssc_o[0] = jnp.concatenate(
        [S * _dinv2_col(cp[0, 1]), jnp.zeros((NG, F - D2), jnp.float32)], axis=1)
    E0_o[0] = jnp.sum(nf2, axis=0, keepdims=True) * (1.0 / 32.0)


def _tc_mid(scat, hs, cp, bg1, Wf1, bf1, Wf2, bf2):
    return pl.pallas_call(
        _tc_mid_body,
        grid=(G,),
        in_specs=[
            pl.BlockSpec((1, NG, F), lambda g: (g, 0, 0)),
            pl.BlockSpec((1, NG, F), lambda g: (g, 0, 0)),
            pl.BlockSpec((1, 2, NS, NG), lambda g: (g, 0, 0, 0)),
            pl.BlockSpec((1, F), lambda g: (0, 0)),
            pl.BlockSpec((F, F), lambda g: (0, 0)),
            pl.BlockSpec((1, F), lambda g: (0, 0)),
            pl.BlockSpec((F, D2), lambda g: (0, 0)),
            pl.BlockSpec((1, D2), lambda g: (0, 0)),
        ],
        out_specs=[
            pl.BlockSpec((1, NG, D2), lambda g: (g, 0, 0)),
            pl.BlockSpec((1, NG, F), lambda g: (g, 0, 0)),
            pl.BlockSpec((1, 1, F), lambda g: (g, 0, 0)),
        ],
        out_shape=[
            jax.ShapeDtypeStruct((G, NG, D2), jnp.float32),
            jax.ShapeDtypeStruct((G, NG, F), jnp.float32),
            jax.ShapeDtypeStruct((G, 1, F), jnp.float32),
        ],
    )(scat, hs, cp, bg1, Wf1, bf1, Wf2, bf2)


def _tc_pen_body(scat2, S_in, cp, pen_o):
    Sg = S_in[0]
    lms = Sg - _dinv2_col(cp[0, 1]) * scat2[0][:, :D2]
    na = lax.dot_general(Sg, lms, (((0,), (0,)), ((), ())),
                         preferred_element_type=jnp.float32)
    rs = jnp.sum(jnp.abs(na), axis=1, keepdims=True)
    eye = (lax.broadcasted_iota(jnp.int32, (D2, D2), 0)
           == lax.broadcasted_iota(jnp.int32, (D2, D2), 1))
    dcol = (jnp.sum(jnp.where(eye, na, 0.0), axis=1, keepdims=True)
            / jnp.maximum(rs, 1e-12))
    pen = jnp.sum(31.0 * dcol * dcol + (dcol - 1.0) ** 2) * (1.0 / 1024.0)
    pen_o[0] = jnp.full((1, 128), pen, jnp.float32)


def _tc_pen(scat2, S, cp):
    return pl.pallas_call(
        _tc_pen_body,
        grid=(G,),
        in_specs=[
            pl.BlockSpec((1, NG, F), lambda g: (g, 0, 0)),
            pl.BlockSpec((1, NG, D2), lambda g: (g, 0, 0)),
            pl.BlockSpec((1, 2, NS, NG), lambda g: (g, 0, 0, 0)),
        ],
        out_specs=[pl.BlockSpec((1, 1, 128), lambda g: (g, 0, 0))],
        out_shape=[jax.ShapeDtypeStruct((G, 1, 128), jnp.float32)],
    )(scat2, S, cp)[0]


def _tc_final_body(E0, pos_e, neg_e, Wc1, bc1, Wmu, bmu, Wls, bls,
                   Wl1, bl1, Wl2, bl2, emb_w, pen_vec,
                   loss_o, pen_o, pp_o, np_o):
    f32 = jnp.float32

    def onehot(col):
        return (col == lax.broadcasted_iota(jnp.int32, (P, NN), 1)).astype(f32)

    Os = onehot(pos_e[:, 0:1])
    Od = onehot(pos_e[:, 1:2])
    Ns = onehot(neg_e[:, 0:1])
    Nd = onehot(neg_e[:, 1:2])

    ones_col = jnp.ones((P, 1), f32)
    cnt = lax.dot_general(Od, ones_col, (((0,), (0,)), ((), ())),
                          preferred_element_type=f32)
    dinv = lax.rsqrt(cnt + 1.0)
    norm = (jnp.dot(Os, dinv, preferred_element_type=f32)
            * jnp.dot(Od, dinv, preferred_element_type=f32))
    d2 = dinv * dinv

    def gcn2(X, W, b):
        H = jnp.dot(X, W[...], preferred_element_type=f32)
        gath = jnp.dot(Os, H, preferred_element_type=f32)
        agg = lax.dot_general(Od, norm * gath, (((0,), (0,)), ((), ())),
                              preferred_element_type=f32)
        return agg + d2 * H + b[...]

    h1 = jax.nn.relu(gcn2(E0[...], Wc1, bc1))
    mu = gcn2(h1, Wmu, bmu)
    ls = jnp.minimum(gcn2(h1, Wls, bls), 10.0)

    emb_full = jnp.concatenate([emb_w[...], mu], axis=1)

    def pred(Oa, Ob):
        fx = jnp.dot(jnp.dot(Oa, emb_full, preferred_element_type=f32),
                     Wl1[...], preferred_element_type=f32) + bl1[...]
        fy = jnp.dot(jnp.dot(Ob, emb_full, preferred_element_type=f32),
                     Wl2[...], preferred_element_type=f32) + bl2[...]
        return jax.nn.sigmoid(-jnp.sum(fx * fy, axis=1, keepdims=True))

    ppred = pred(Os, Od)
    npred = pred(Ns, Nd)
    EPS = 1e-15
    rec = (-jnp.mean(jnp.log(ppred + EPS))
           - jnp.mean(jnp.log(1.0 - npred + EPS)))
    kl = -0.5 * jnp.sum(1.0 + 2.0 * ls - mu * mu - jnp.exp(2.0 * ls)) \
        * (1.0 / (NN * NN))
    loss_o[...] = jnp.full((1, 1), rec + kl, f32)
    pen_o[...] = jnp.full((1, 1), jnp.sum(pen_vec[:, :1]) * (1.0 / G), f32)
    pp_o[...] = ppred
    np_o[...] = npred


def _tc_final(E0, pos_e, neg_e, Wc1, bc1, Wmu, bmu, Wls, bls,
              Wl1, bl1, Wl2, bl2, emb_w, pen_vec):
    return pl.pallas_call(
        _tc_final_body,
        out_shape=[
            jax.ShapeDtypeStruct((1, 1), jnp.float32),
            jax.ShapeDtypeStruct((1, 1), jnp.float32),
            jax.ShapeDtypeStruct((P, 1), jnp.float32),
            jax.ShapeDtypeStruct((P, 1), jnp.float32),
        ],
    )(E0, pos_e, neg_e, Wc1, bc1, Wmu, bmu, Wls, bls,
      Wl1, bl1, Wl2, bl2, emb_w, pen_vec)


def kernel(features, edges, pos_edges, neg_edges, Wg1, bg1, Wf1, bf1,
           Wf2, bf2, Wc1, bc1, Wmu, bmu, Wls, bls, Wl1, bl1, Wl2, bl2,
           emb_w):
    er = edges.astype(jnp.int32).reshape(G, 2, NS, NCH, 128)

    cp = _sc_counts(er).reshape(G, 2, NS, NG)

    hs, srcg, dstg = _tc_prep(features, Wg1, cp, er)

    scat = _sc_scatter(hs.reshape(G * NG, F), srcg,
                       er[:, 1], width=F)

    S, Ssc, E0 = _tc_mid(scat, hs, cp, bg1.reshape(1, F),
                         Wf1, bf1.reshape(1, F), Wf2, bf2.reshape(1, D2))
    E0 = E0.reshape(G, F)

    scat2 = _sc_scatter(Ssc.reshape(G * NG, F), dstg,
                        er[:, 0], width=F)

    pen_vec = _tc_pen(scat2, S, cp).reshape(G, 128)

    loss, pen, pp, npred = _tc_final(
        E0, pos_edges.astype(jnp.int32), neg_edges.astype(jnp.int32),
        Wc1, bc1.reshape(1, 2 * DG), Wmu, bmu.reshape(1, DG),
        Wls, bls.reshape(1, DG), Wl1, bl1.reshape(1, F),
        Wl2, bl2.reshape(1, F), emb_w, pen_vec)

    return (loss[0, 0], pen[0, 0], pp[:, 0], npred[:, 0])


# speedup vs baseline: 30.8677x; 1.0595x over previous
import functools

import jax
import jax.numpy as jnp
from jax import lax
from jax.experimental import pallas as pl
from jax.experimental.pallas import tpu as pltpu
from jax.experimental.pallas import tpu_sc as plsc

G, NG, EG, F = 32, 1024, 16384, 128
D2 = 32
DG = 64
NN = 32
P = 512

NC, NS = 2, 16
GPC = G // NC
EPT = EG // NS
NCH = EPT // 128
RPT = NG // NS

_MESH = dict(core_axis_name="c", subcore_axis_name="s", num_cores=NC,
             num_subcores=NS)


def _sc_counts_body(gpc, er, z_hbm, out, idx_v, cntd, cnts):
    c = lax.axis_index("c")
    s = lax.axis_index("s")
    one16 = jnp.ones((16,), jnp.float32)

    def body(gi):
        g = c * gpc + gi
        pltpu.sync_copy(z_hbm, cntd)
        pltpu.sync_copy(z_hbm, cnts)
        pltpu.sync_copy(er.at[g, 1, s], idx_v)
        for j in range(NCH):
            for k in range(8):
                idx = idx_v[j, pl.ds(k * 16, 16)]
                plsc.addupdate_scatter(cntd, [idx], one16)
        pltpu.sync_copy(er.at[g, 0, s], idx_v)
        for j in range(NCH):
            for k in range(8):
                idx = idx_v[j, pl.ds(k * 16, 16)]
                plsc.addupdate_scatter(cnts, [idx], one16)
        pltpu.sync_copy(cntd, out.at[g, 0, pl.ds(s * NG, NG)])
        pltpu.sync_copy(cnts, out.at[g, 1, pl.ds(s * NG, NG)])

    pl.loop(0, gpc)(body)


def _sc_counts(er):
    gb = er.shape[0]
    mesh = plsc.VectorSubcoreMesh(**_MESH)
    zeros = jnp.zeros((NG,), jnp.float32)
    fn = pl.kernel(
        functools.partial(_sc_counts_body, gb // NC),
        out_type=jax.ShapeDtypeStruct((gb, 2, NS * NG), jnp.float32),
        mesh=mesh,
        scratch_types=[
            pltpu.VMEM((NCH, 128), jnp.int32),
            pltpu.VMEM((NG,), jnp.float32),
            pltpu.VMEM((NG,), jnp.float32),
        ],
        compiler_params=pltpu.CompilerParams(needs_layout_passes=False),
    )
    return fn(er, zeros)


def _sc_scatter_body(width, gpc, table, gidx, sidx, z_hbm, out,
                     idx_g, idx_s, rb0, rb1, rb2, zbuf, acc, semg, sems):
    c = lax.axis_index("c")
    s = lax.axis_index("s")
    base = s * RPT
    pltpu.sync_copy(z_hbm, zbuf)
    rbs = (rb0, rb1, rb2)
    for gi in range(gpc):
        g = c * gpc + gi
        pltpu.sync_copy(zbuf, acc.at[pl.ds(base, RPT)])
        pltpu.sync_copy(gidx.at[g, s], idx_g)
        pltpu.sync_copy(sidx.at[g, s], idx_s)
        plsc.subcore_barrier()
        cps = [None] * NCH
        scs = [None] * NCH
        cps[0] = pltpu.async_copy(table.at[idx_g.at[0]], rbs[0], semg)
        if NCH > 1:
            cps[1] = pltpu.async_copy(table.at[idx_g.at[1]], rbs[1], semg)
        for j in range(NCH):
            cps[j].wait()
            if j >= 1:
                scs[j - 1].wait()
            if j + 2 < NCH:
                cps[j + 2] = pltpu.async_copy(table.at[idx_g.at[j + 2]],
                                              rbs[(j + 2) % 3], semg)
            scs[j] = pltpu.async_copy(rbs[j % 3], acc.at[idx_s.at[j]],
                                      sems, add=True)
        scs[NCH - 1].wait()
        plsc.subcore_barrier()
        pltpu.sync_copy(acc.at[pl.ds(base, RPT)],
                        out.at[g, pl.ds(base, RPT)])
        plsc.subcore_barrier()


def _sc_scatter(table, gidx, sidx, width):
    gb = gidx.shape[0]
    mesh = plsc.VectorSubcoreMesh(**_MESH)
    zeros = jnp.zeros((RPT, width), jnp.float32)
    fn = pl.kernel(
        functools.partial(_sc_scatter_body, width, gb // NC),
        out_type=jax.ShapeDtypeStruct((gb, NG, width), jnp.float32),
        mesh=mesh,
        scratch_types=[
            pltpu.VMEM((NCH, 128), jnp.int32),
            pltpu.VMEM((NCH, 128), jnp.int32),
            pltpu.VMEM((128, width), jnp.float32),
            pltpu.VMEM((128, width), jnp.float32),
            pltpu.VMEM((128, width), jnp.float32),
            pltpu.VMEM((RPT, width), jnp.float32),
            pltpu.VMEM_SHARED((NG, width), jnp.float32),
            pltpu.SemaphoreType.DMA,
            pltpu.SemaphoreType.DMA,
        ],
    )
    return fn(table, gidx, sidx, zeros)


def _cnt_col(part):
    ones = jnp.ones((NS, 1), jnp.float32)
    return lax.dot_general(part, ones, (((0,), (0,)), ((), ())),
                           preferred_element_type=jnp.float32)


def _dinv2_col(cs_part):
    col = _cnt_col(cs_part)
    return jnp.where(col > 0.0, lax.rsqrt(jnp.maximum(col, 1.0)), 0.0)


def _tc_prep_body(feat, Wg1, cp, er, hs_o, srcg_o, dstg_o):
    g = pl.program_id(0)
    h = jnp.dot(feat[0], Wg1[...], preferred_element_type=jnp.float32)
    dinv = lax.rsqrt(_cnt_col(cp[0, 0]) + 1.0)
    hs_o[0] = h * dinv
    off = g * NG
    srcg_o[0] = er[0, 0] + off
    dstg_o[0] = er[0, 1] + off


def _tc_prep(features, Wg1, cd, er):
    gb = features.shape[0]
    return pl.pallas_call(
        _tc_prep_body,
        grid=(gb,),
        in_specs=[
            pl.BlockSpec((1, NG, F), lambda g: (g, 0, 0)),
            pl.BlockSpec((F, F), lambda g: (0, 0)),
            pl.BlockSpec((1, 2, NS, NG), lambda g: (g, 0, 0, 0)),
            pl.BlockSpec((1, 2, NS, NCH, 128), lambda g: (g, 0, 0, 0, 0)),
        ],
        out_specs=[
            pl.BlockSpec((1, NG, F), lambda g: (g, 0, 0)),
            pl.BlockSpec((1, NS, NCH, 128), lambda g: (g, 0, 0, 0)),
            pl.BlockSpec((1, NS, NCH, 128), lambda g: (g, 0, 0, 0)),
        ],
        out_shape=[
            jax.ShapeDtypeStruct((gb, NG, F), jnp.float32),
            jax.ShapeDtypeStruct((gb, NS, NCH, 128), jnp.int32),
            jax.ShapeDtypeStruct((gb, NS, NCH, 128), jnp.int32),
        ],
    )(features, Wg1, cd, er)


def _tc_mid_body(scat, hs, cp, bg1, Wf1, bf1, Wf2, bf2,
                 S_o, Ssc_o, E0_o):
    dinv = lax.rsqrt(_cnt_col(cp[0, 0]) + 1.0)
    nf2 = (scat[0] + hs[0]) * dinv + bg1[...]
    a1 = jnp.tanh(jnp.dot(nf2, Wf1[...], preferred_element_type=jnp.float32)
                  + bf1[...])
    logits = (jnp.dot(a1, Wf2[...], preferred_element_type=jnp.float32)
              + bf2[...])
    S = jax.nn.softmax(logits, axis=1)
    S_o[0] = S
    Ssc_o[0] = jnp.concatenate(
        [S * _dinv2_col(cp[0, 1]), jnp.zeros((NG, F - D2), jnp.float32)], axis=1)
    E0_o[0] = jnp.sum(nf2, axis=0, keepdims=True) * (1.0 / 32.0)


def _tc_mid(scat, hs, cp, bg1, Wf1, bf1, Wf2, bf2):
    gb = scat.shape[0]
    return pl.pallas_call(
        _tc_mid_body,
        grid=(gb,),
        in_specs=[
            pl.BlockSpec((1, NG, F), lambda g: (g, 0, 0)),
            pl.BlockSpec((1, NG, F), lambda g: (g, 0, 0)),
            pl.BlockSpec((1, 2, NS, NG), lambda g: (g, 0, 0, 0)),
            pl.BlockSpec((1, F), lambda g: (0, 0)),
            pl.BlockSpec((F, F), lambda g: (0, 0)),
            pl.BlockSpec((1, F), lambda g: (0, 0)),
            pl.BlockSpec((F, D2), lambda g: (0, 0)),
            pl.BlockSpec((1, D2), lambda g: (0, 0)),
        ],
        out_specs=[
            pl.BlockSpec((1, NG, D2), lambda g: (g, 0, 0)),
            pl.BlockSpec((1, NG, F), lambda g: (g, 0, 0)),
            pl.BlockSpec((1, 1, F), lambda g: (g, 0, 0)),
        ],
        out_shape=[
            jax.ShapeDtypeStruct((gb, NG, D2), jnp.float32),
            jax.ShapeDtypeStruct((gb, NG, F), jnp.float32),
            jax.ShapeDtypeStruct((gb, 1, F), jnp.float32),
        ],
    )(scat, hs, cp, bg1, Wf1, bf1, Wf2, bf2)


def _tc_pen_body(scat2, S_in, cp, pen_o):
    Sg = S_in[0]
    lms = Sg - _dinv2_col(cp[0, 1]) * scat2[0][:, :D2]
    na = lax.dot_general(Sg, lms, (((0,), (0,)), ((), ())),
                         preferred_element_type=jnp.float32)
    rs = jnp.sum(jnp.abs(na), axis=1, keepdims=True)
    eye = (lax.broadcasted_iota(jnp.int32, (D2, D2), 0)
           == lax.broadcasted_iota(jnp.int32, (D2, D2), 1))
    dcol = (jnp.sum(jnp.where(eye, na, 0.0), axis=1, keepdims=True)
            / jnp.maximum(rs, 1e-12))
    pen = jnp.sum(31.0 * dcol * dcol + (dcol - 1.0) ** 2) * (1.0 / 1024.0)
    pen_o[0] = jnp.full((1, 128), pen, jnp.float32)


def _tc_pen(scat2, S, cp):
    gb = scat2.shape[0]
    return pl.pallas_call(
        _tc_pen_body,
        grid=(gb,),
        in_specs=[
            pl.BlockSpec((1, NG, F), lambda g: (g, 0, 0)),
            pl.BlockSpec((1, NG, D2), lambda g: (g, 0, 0)),
            pl.BlockSpec((1, 2, NS, NG), lambda g: (g, 0, 0, 0)),
        ],
        out_specs=[pl.BlockSpec((1, 1, 128), lambda g: (g, 0, 0))],
        out_shape=[jax.ShapeDtypeStruct((gb, 1, 128), jnp.float32)],
    )(scat2, S, cp)[0]


def _tc_final_body(E0, pos_e, neg_e, Wc1, bc1, Wmu, bmu, Wls, bls,
                   Wl1, bl1, Wl2, bl2, emb_w, pen_vec,
                   loss_o, pen_o, pp_o, np_o):
    f32 = jnp.float32

    def onehot(col):
        return (col == lax.broadcasted_iota(jnp.int32, (P, NN), 1)).astype(f32)

    Os = onehot(pos_e[:, 0:1])
    Od = onehot(pos_e[:, 1:2])
    Ns = onehot(neg_e[:, 0:1])
    Nd = onehot(neg_e[:, 1:2])

    ones_col = jnp.ones((P, 1), f32)
    cnt = lax.dot_general(Od, ones_col, (((0,), (0,)), ((), ())),
                          preferred_element_type=f32)
    dinv = lax.rsqrt(cnt + 1.0)
    norm = (jnp.dot(Os, dinv, preferred_element_type=f32)
            * jnp.dot(Od, dinv, preferred_element_type=f32))
    d2 = dinv * dinv

    def gcn2(X, W, b):
        H = jnp.dot(X, W[...], preferred_element_type=f32)
        gath = jnp.dot(Os, H, preferred_element_type=f32)
        agg = lax.dot_general(Od, norm * gath, (((0,), (0,)), ((), ())),
                              preferred_element_type=f32)
        return agg + d2 * H + b[...]

    h1 = jax.nn.relu(gcn2(E0[...], Wc1, bc1))
    mu = gcn2(h1, Wmu, bmu)
    ls = jnp.minimum(gcn2(h1, Wls, bls), 10.0)

    emb_full = jnp.concatenate([emb_w[...], mu], axis=1)

    def pred(Oa, Ob):
        fx = jnp.dot(jnp.dot(Oa, emb_full, preferred_element_type=f32),
                     Wl1[...], preferred_element_type=f32) + bl1[...]
        fy = jnp.dot(jnp.dot(Ob, emb_full, preferred_element_type=f32),
                     Wl2[...], preferred_element_type=f32) + bl2[...]
        return jax.nn.sigmoid(-jnp.sum(fx * fy, axis=1, keepdims=True))

    ppred = pred(Os, Od)
    npred = pred(Ns, Nd)
    EPS = 1e-15
    rec = (-jnp.mean(jnp.log(ppred + EPS))
           - jnp.mean(jnp.log(1.0 - npred + EPS)))
    kl = -0.5 * jnp.sum(1.0 + 2.0 * ls - mu * mu - jnp.exp(2.0 * ls)) \
        * (1.0 / (NN * NN))
    loss_o[...] = jnp.full((1, 1), rec + kl, f32)
    pen_o[...] = jnp.full((1, 1), jnp.sum(pen_vec[:, :1]) * (1.0 / G), f32)
    pp_o[...] = ppred
    np_o[...] = npred


def _tc_final(E0, pos_e, neg_e, Wc1, bc1, Wmu, bmu, Wls, bls,
              Wl1, bl1, Wl2, bl2, emb_w, pen_vec):
    return pl.pallas_call(
        _tc_final_body,
        out_shape=[
            jax.ShapeDtypeStruct((1, 1), jnp.float32),
            jax.ShapeDtypeStruct((1, 1), jnp.float32),
            jax.ShapeDtypeStruct((P, 1), jnp.float32),
            jax.ShapeDtypeStruct((P, 1), jnp.float32),
        ],
    )(E0, pos_e, neg_e, Wc1, bc1, Wmu, bmu, Wls, bls,
      Wl1, bl1, Wl2, bl2, emb_w, pen_vec)


def kernel(features, edges, pos_edges, neg_edges, Wg1, bg1, Wf1, bf1,
           Wf2, bf2, Wc1, bc1, Wmu, bmu, Wls, bls, Wl1, bl1, Wl2, bl2,
           emb_w):
    er = edges.astype(jnp.int32).reshape(G, 2, NS, NCH, 128)

    NB = 2
    GB = G // NB
    ers = [er[b * GB:(b + 1) * GB] for b in range(NB)]
    feats = [features[b * GB:(b + 1) * GB] for b in range(NB)]

    cps = [_sc_counts(e).reshape(GB, 2, NS, NG) for e in ers]
    preps = [_tc_prep(feats[b], Wg1, cps[b], ers[b]) for b in range(NB)]
    scats = [_sc_scatter(preps[b][0].reshape(GB * NG, F), preps[b][1],
                         ers[b][:, 1], width=F) for b in range(NB)]
    mids = [_tc_mid(scats[b], preps[b][0], cps[b], bg1.reshape(1, F),
                    Wf1, bf1.reshape(1, F), Wf2, bf2.reshape(1, D2))
            for b in range(NB)]
    scat2s = [_sc_scatter(mids[b][1].reshape(GB * NG, F), preps[b][2],
                          ers[b][:, 0], width=F) for b in range(NB)]
    pens = [_tc_pen(scat2s[b], mids[b][0], cps[b]).reshape(GB, 128)
            for b in range(NB)]

    E0 = jnp.concatenate([m[2].reshape(GB, F) for m in mids], axis=0)
    pen_vec = jnp.concatenate(pens, axis=0)

    loss, pen, pp, npred = _tc_final(
        E0, pos_edges.astype(jnp.int32), neg_edges.astype(jnp.int32),
        Wc1, bc1.reshape(1, 2 * DG), Wmu, bmu.reshape(1, DG),
        Wls, bls.reshape(1, DG), Wl1, bl1.reshape(1, F),
        Wl2, bl2.reshape(1, F), emb_w, pen_vec)

    return (loss[0, 0], pen[0, 0], pp[:, 0], npred[:, 0])
